# baseline (speedup 1.0000x reference)
import jax
import jax.numpy as jnp
from jax.experimental import pallas as pl
from jax.experimental.pallas import tpu as pltpu
from jax.experimental.pallas import tpu_sc as plsc

_W = 128
_SPLIT = 2

_TC_ROWS = 2048
_SC_BLOCKS = 64


def _sc_gather_prefix(idx_t, table, n_total):
    n = idx_t.shape[0]
    D = table.shape[1]
    Dh = D // _SPLIT
    idx2 = (
        _SPLIT * idx_t.reshape(1, -1)
        + jnp.arange(_SPLIT, dtype=idx_t.dtype).reshape(-1, 1)
    )
    tab = table.reshape(-1, Dh)

    mesh = plsc.VectorSubcoreMesh(
        core_axis_name="core", subcore_axis_name="subcore"
    )

    @pl.kernel(
        out_type=jax.ShapeDtypeStruct((n_total, D), jnp.float32),
        mesh=mesh,
    )
    def run(tab_hbm, idx_hbm, out_hbm):
        def body(i_vmem, o_vmem):
            pltpu.sync_copy(tab_hbm.at[i_vmem.at[0]], o_vmem)

        pltpu.emit_pipeline(
            body,
            grid=(n // _W, _SPLIT),
            in_specs=[pl.BlockSpec((1, _W), index_map=lambda i, j: (j, i))],
            out_specs=[pl.BlockSpec((_W, Dh), index_map=lambda i, j: (i, j))],
            core_axis_name=("core", "subcore"),
            dimension_semantics=(pltpu.PARALLEL, pltpu.PARALLEL),
        )(idx_hbm, out_hbm)

    return run(tab, idx2)


def _onehot(idx_t, v_pad):
    return (
        idx_t.reshape(-1, 1)
        == (jnp.arange(2 * v_pad, dtype=jnp.int32) % v_pad).reshape(1, -1)
    ).astype(jnp.int8)


def _tc_matmul_body(v_pad):
    def body(oh_ref, t_ref, o_ref, hl_ref):
        @pl.when(pl.program_id(0) == 0)
        def _():
            t = t_ref[...]
            hi = t.astype(jnp.bfloat16)
            hl_ref[:v_pad] = hi
            hl_ref[v_pad:] = (t - hi.astype(jnp.float32)).astype(jnp.bfloat16)

        o_ref[...] = jax.lax.dot(
            oh_ref[...].astype(jnp.bfloat16),
            hl_ref[...],
            preferred_element_type=jnp.float32,
        )

    return body


def _tc_lookup(idx_t, table, v_pad):
    N = idx_t.shape[0]
    D = table.shape[1]
    steps = N // _TC_ROWS
    tab_pad = jnp.zeros((v_pad, D), jnp.float32).at[: table.shape[0]].set(table)

    return pl.pallas_call(
        _tc_matmul_body(v_pad),
        grid=(steps,),
        in_specs=[
            pl.BlockSpec((_TC_ROWS, 2 * v_pad), lambda i: (i, 0)),
            pl.BlockSpec((v_pad, D), lambda i: (0, 0)),
        ],
        out_specs=pl.BlockSpec((_TC_ROWS, D), lambda i: (i, 0)),
        out_shape=jax.ShapeDtypeStruct((N, D), jnp.float32),
        scratch_shapes=[pltpu.VMEM((2 * v_pad, D), jnp.bfloat16)],
    )(_onehot(idx_t, v_pad), tab_pad)


def _tc_finish(partial, idx_t_tail, table, v_pad, k0):
    N, D = partial.shape
    steps = N // _TC_ROWS - k0
    tab_pad = jnp.zeros((v_pad, D), jnp.float32).at[: table.shape[0]].set(table)

    def body(p_ref, oh_ref, t_ref, o_ref, hl_ref):
        del p_ref
        _tc_matmul_body(v_pad)(oh_ref, t_ref, o_ref, hl_ref)

    return pl.pallas_call(
        body,
        grid=(steps,),
        in_specs=[
            pl.BlockSpec(memory_space=pl.ANY),
            pl.BlockSpec((_TC_ROWS, 2 * v_pad), lambda i: (i, 0)),
            pl.BlockSpec((v_pad, D), lambda i: (0, 0)),
        ],
        out_specs=pl.BlockSpec((_TC_ROWS, D), lambda i: (i + k0, 0)),
        out_shape=jax.ShapeDtypeStruct((N, D), jnp.float32),
        scratch_shapes=[pltpu.VMEM((2 * v_pad, D), jnp.bfloat16)],
        input_output_aliases={0: 0},
    )(partial, _onehot(idx_t_tail, v_pad), tab_pad)


def kernel(node_idx, edge_idx, node_table, edge_table):
    B, S = node_idx.shape
    D = node_table.shape[1]
    n_idx, e_idx = jax.lax.optimization_barrier(
        (node_idx.T.reshape(-1), edge_idx.T.reshape(-1))
    )

    n_sc = _SC_BLOCKS * _TC_ROWS
    partial = _sc_gather_prefix(n_idx[:n_sc], node_table, n_idx.shape[0])
    node_out = _tc_finish(partial, n_idx[n_sc:], node_table, 256, _SC_BLOCKS)
    edge_out = _tc_lookup(e_idx, edge_table, 128)

    node_out = node_out.reshape(S, B, D).transpose(1, 0, 2)
    edge_out = edge_out.reshape(S, B, D).transpose(1, 0, 2)
    return node_out, edge_out


# speedup vs baseline: 1.0003x; 1.0003x over previous
import jax
import jax.numpy as jnp
from jax.experimental import pallas as pl
from jax.experimental.pallas import tpu as pltpu
from jax.experimental.pallas import tpu_sc as plsc

_W = 128
_SPLIT = 2

_TC_ROWS = 2048
_SC_BLOCKS = 70


def _sc_gather_prefix(idx_t, table, n_total):
    n = idx_t.shape[0]
    D = table.shape[1]
    Dh = D // _SPLIT
    idx2 = (
        _SPLIT * idx_t.reshape(1, -1)
        + jnp.arange(_SPLIT, dtype=idx_t.dtype).reshape(-1, 1)
    )
    tab = table.reshape(-1, Dh)

    mesh = plsc.VectorSubcoreMesh(
        core_axis_name="core", subcore_axis_name="subcore"
    )

    @pl.kernel(
        out_type=jax.ShapeDtypeStruct((n_total, D), jnp.float32),
        mesh=mesh,
    )
    def run(tab_hbm, idx_hbm, out_hbm):
        def body(i_vmem, o_vmem):
            pltpu.sync_copy(tab_hbm.at[i_vmem.at[0]], o_vmem)

        pltpu.emit_pipeline(
            body,
            grid=(n // _W, _SPLIT),
            in_specs=[pl.BlockSpec((1, _W), index_map=lambda i, j: (j, i))],
            out_specs=[pl.BlockSpec((_W, Dh), index_map=lambda i, j: (i, j))],
            core_axis_name=("core", "subcore"),
            dimension_semantics=(pltpu.PARALLEL, pltpu.PARALLEL),
        )(idx_hbm, out_hbm)

    return run(tab, idx2)


def _onehot(idx_t, v_pad):
    return (
        idx_t.reshape(-1, 1)
        == (jnp.arange(2 * v_pad, dtype=jnp.int32) % v_pad).reshape(1, -1)
    ).astype(jnp.int8)


def _tc_matmul_body(v_pad):
    def body(oh_ref, t_ref, o_ref, hl_ref):
        @pl.when(pl.program_id(0) == 0)
        def _():
            t = t_ref[...]
            hi = t.astype(jnp.bfloat16)
            hl_ref[:v_pad] = hi
            hl_ref[v_pad:] = (t - hi.astype(jnp.float32)).astype(jnp.bfloat16)

        o_ref[...] = jax.lax.dot(
            oh_ref[...].astype(jnp.bfloat16),
            hl_ref[...],
            preferred_element_type=jnp.float32,
        )

    return body


def _tc_lookup(idx_t, table, v_pad):
    N = idx_t.shape[0]
    D = table.shape[1]
    steps = N // _TC_ROWS
    tab_pad = jnp.zeros((v_pad, D), jnp.float32).at[: table.shape[0]].set(table)

    return pl.pallas_call(
        _tc_matmul_body(v_pad),
        grid=(steps,),
        in_specs=[
            pl.BlockSpec((_TC_ROWS, 2 * v_pad), lambda i: (i, 0)),
            pl.BlockSpec((v_pad, D), lambda i: (0, 0)),
        ],
        out_specs=pl.BlockSpec((_TC_ROWS, D), lambda i: (i, 0)),
        out_shape=jax.ShapeDtypeStruct((N, D), jnp.float32),
        scratch_shapes=[pltpu.VMEM((2 * v_pad, D), jnp.bfloat16)],
    )(_onehot(idx_t, v_pad), tab_pad)


def _tc_finish(partial, idx_t_tail, table, v_pad, k0):
    N, D = partial.shape
    steps = N // _TC_ROWS - k0
    tab_pad = jnp.zeros((v_pad, D), jnp.float32).at[: table.shape[0]].set(table)

    def body(p_ref, oh_ref, t_ref, o_ref, hl_ref):
        del p_ref
        _tc_matmul_body(v_pad)(oh_ref, t_ref, o_ref, hl_ref)

    return pl.pallas_call(
        body,
        grid=(steps,),
        in_specs=[
            pl.BlockSpec(memory_space=pl.ANY),
            pl.BlockSpec((_TC_ROWS, 2 * v_pad), lambda i: (i, 0)),
            pl.BlockSpec((v_pad, D), lambda i: (0, 0)),
        ],
        out_specs=pl.BlockSpec((_TC_ROWS, D), lambda i: (i + k0, 0)),
        out_shape=jax.ShapeDtypeStruct((N, D), jnp.float32),
        scratch_shapes=[pltpu.VMEM((2 * v_pad, D), jnp.bfloat16)],
        input_output_aliases={0: 0},
    )(partial, _onehot(idx_t_tail, v_pad), tab_pad)


def kernel(node_idx, edge_idx, node_table, edge_table):
    B, S = node_idx.shape
    D = node_table.shape[1]
    n_idx = node_idx.T.reshape(-1)
    e_idx = edge_idx.T.reshape(-1)

    n_sc = _SC_BLOCKS * _TC_ROWS
    partial = _sc_gather_prefix(n_idx[:n_sc], node_table, n_idx.shape[0])
    node_out = _tc_finish(partial, n_idx[n_sc:], node_table, 256, _SC_BLOCKS)
    edge_out = _tc_lookup(e_idx, edge_table, 128)

    node_out = node_out.reshape(S, B, D).transpose(1, 0, 2)
    edge_out = edge_out.reshape(S, B, D).transpose(1, 0, 2)
    return node_out, edge_out


# speedup vs baseline: 1.0883x; 1.0879x over previous
import jax
import jax.numpy as jnp
from jax.experimental import pallas as pl
from jax.experimental.pallas import tpu as pltpu
from jax.experimental.pallas import tpu_sc as plsc

_W = 128
_SPLIT = 2

_TC_ROWS = 2048
_SC_BLOCKS = 58


def _sc_gather_prefix(idx_t, table, n_total):
    n = idx_t.shape[0]
    D = table.shape[1]
    Dh = D // _SPLIT
    idx2 = (
        _SPLIT * idx_t.reshape(1, -1)
        + jnp.arange(_SPLIT, dtype=idx_t.dtype).reshape(-1, 1)
    )
    tab = table.reshape(-1, Dh)

    mesh = plsc.VectorSubcoreMesh(
        core_axis_name="core", subcore_axis_name="subcore"
    )

    @pl.kernel(
        out_type=jax.ShapeDtypeStruct((n_total, D), jnp.float32),
        mesh=mesh,
    )
    def run(tab_hbm, idx_hbm, out_hbm):
        def body(i_vmem, o_vmem):
            pltpu.sync_copy(tab_hbm.at[i_vmem.at[0]], o_vmem)

        pltpu.emit_pipeline(
            body,
            grid=(n // _W, _SPLIT),
            in_specs=[pl.BlockSpec((1, _W), index_map=lambda i, j: (j, i))],
            out_specs=[pl.BlockSpec((_W, Dh), index_map=lambda i, j: (i, j))],
            core_axis_name=("core", "subcore"),
            dimension_semantics=(pltpu.PARALLEL, pltpu.PARALLEL),
        )(idx_hbm, out_hbm)

    return run(tab, idx2)


def _onehot(idx_t, v_pad):
    return (
        idx_t.reshape(-1, 1)
        == (jnp.arange(2 * v_pad, dtype=jnp.int32) % v_pad).reshape(1, -1)
    ).astype(jnp.int8)


def _tc_matmul_body(v_pad):
    def body(oh_ref, t_ref, o_ref, hl_ref):
        @pl.when(pl.program_id(0) == 0)
        def _():
            t = t_ref[...]
            hi = t.astype(jnp.bfloat16)
            hl_ref[:v_pad] = hi
            hl_ref[v_pad:] = (t - hi.astype(jnp.float32)).astype(jnp.bfloat16)

        o_ref[...] = jax.lax.dot(
            oh_ref[...].astype(jnp.bfloat16),
            hl_ref[...],
            preferred_element_type=jnp.float32,
        )

    return body


def _tc_lookup(idx_t, table, v_pad):
    N = idx_t.shape[0]
    D = table.shape[1]
    steps = N // _TC_ROWS
    tab_pad = jnp.zeros((v_pad, D), jnp.float32).at[: table.shape[0]].set(table)

    return pl.pallas_call(
        _tc_matmul_body(v_pad),
        grid=(steps,),
        in_specs=[
            pl.BlockSpec((_TC_ROWS, 2 * v_pad), lambda i: (i, 0)),
            pl.BlockSpec((v_pad, D), lambda i: (0, 0)),
        ],
        out_specs=pl.BlockSpec((_TC_ROWS, D), lambda i: (i, 0)),
        out_shape=jax.ShapeDtypeStruct((N, D), jnp.float32),
        scratch_shapes=[pltpu.VMEM((2 * v_pad, D), jnp.bfloat16)],
    )(_onehot(idx_t, v_pad), tab_pad)


def _tc_finish(partial, idx_t_tail, table, v_pad, k0):
    N, D = partial.shape
    steps = N // _TC_ROWS - k0
    tab_pad = jnp.zeros((v_pad, D), jnp.float32).at[: table.shape[0]].set(table)

    def body(p_ref, oh_ref, t_ref, o_ref, hl_ref):
        del p_ref
        _tc_matmul_body(v_pad)(oh_ref, t_ref, o_ref, hl_ref)

    return pl.pallas_call(
        body,
        grid=(steps,),
        in_specs=[
            pl.BlockSpec(memory_space=pl.ANY),
            pl.BlockSpec((_TC_ROWS, 2 * v_pad), lambda i: (i, 0)),
            pl.BlockSpec((v_pad, D), lambda i: (0, 0)),
        ],
        out_specs=pl.BlockSpec((_TC_ROWS, D), lambda i: (i + k0, 0)),
        out_shape=jax.ShapeDtypeStruct((N, D), jnp.float32),
        scratch_shapes=[pltpu.VMEM((2 * v_pad, D), jnp.bfloat16)],
        input_output_aliases={0: 0},
    )(partial, _onehot(idx_t_tail, v_pad), tab_pad)


def kernel(node_idx, edge_idx, node_table, edge_table):
    B, S = node_idx.shape
    D = node_table.shape[1]
    n_idx = node_idx.T.reshape(-1)
    e_idx = edge_idx.T.reshape(-1)

    n_sc = _SC_BLOCKS * _TC_ROWS
    partial = _sc_gather_prefix(n_idx[:n_sc], node_table, n_idx.shape[0])
    node_out = _tc_finish(partial, n_idx[n_sc:], node_table, 256, _SC_BLOCKS)
    edge_out = _tc_lookup(e_idx, edge_table, 128)

    node_out = node_out.reshape(S, B, D).transpose(1, 0, 2)
    edge_out = edge_out.reshape(S, B, D).transpose(1, 0, 2)
    return node_out, edge_out
